# baseline (device time: 44604 ns/iter reference)
import functools

import jax
import jax.numpy as jnp
from jax import lax
from jax.experimental import pallas as pl
from jax.experimental.pallas import tpu as pltpu

N_DEV = 8
B = 2
S_LOC = 128
D = 512
HQ = 4
DH = 64
HD = HQ * DH
R = B * S_LOC


def kernel(x, Wq, Wk, Wv, Wo):
    my = lax.axis_index("i")

    pos = (my * S_LOC).astype(jnp.float32) + jnp.arange(S_LOC, dtype=jnp.float32)
    inv = 1.0 / (10000.0 ** (jnp.arange(0, DH, 2, dtype=jnp.float32) / DH))
    ang = pos[:, None] * inv[None, :]
    cos = jnp.repeat(jnp.cos(ang), 2, axis=1)
    sin = jnp.repeat(jnp.sin(ang), 2, axis=1)
    cos_full = jnp.tile(cos, (B, HQ))
    sin_full = jnp.tile(sin, (B, HQ))

    def body(x_ref, wq_ref, wk_ref, wv_ref, wo_ref, cos_ref, sin_ref,
             out_ref, kv_buf, send_sems, recv_sems):
        me = lax.axis_index("i")
        left = lax.rem(me - 1 + N_DEV, N_DEV)
        right = lax.rem(me + 1, N_DEV)

        barrier_sem = pltpu.get_barrier_semaphore()
        for nbr in (left, right):
            pl.semaphore_signal(
                barrier_sem, inc=1,
                device_id=(nbr,), device_id_type=pl.DeviceIdType.MESH,
            )
        pl.semaphore_wait(barrier_sem, 2)

        xf = jnp.concatenate([x_ref[0], x_ref[1]], axis=0).astype(jnp.bfloat16)
        wq = wq_ref[...].astype(jnp.bfloat16)
        wk = wk_ref[...].astype(jnp.bfloat16)
        wv = wv_ref[...].astype(jnp.bfloat16)
        q = jnp.dot(xf, wq, preferred_element_type=jnp.float32)
        k = jnp.dot(xf, wk, preferred_element_type=jnp.float32)
        v = jnp.dot(xf, wv, preferred_element_type=jnp.float32)

        row = lax.broadcasted_iota(jnp.int32, (HD, HD), 0)
        col = lax.broadcasted_iota(jnp.int32, (HD, HD), 1)
        even = (col % 2) == 0
        r_mat = jnp.where(even & (row == col + 1), -1.0, 0.0) + jnp.where(
            (~even) & (row == col - 1), 1.0, 0.0
        )
        cosf = cos_ref[...]
        sinf = sin_ref[...]
        q_rot = q * cosf + jnp.dot(q, r_mat, preferred_element_type=jnp.float32) * sinf
        k_rot = k * cosf + jnp.dot(k, r_mat, preferred_element_type=jnp.float32) * sinf
        qb = q_rot.astype(jnp.bfloat16)

        kv_buf[me, 0] = k_rot.astype(jnp.bfloat16)
        kv_buf[me, 1] = v.astype(jnp.bfloat16)

        for h in range(N_DEV - 1):
            slot = lax.rem(me - h + N_DEV, N_DEV)
            rdma = pltpu.make_async_remote_copy(
                src_ref=kv_buf.at[slot],
                dst_ref=kv_buf.at[slot],
                send_sem=send_sems.at[h],
                recv_sem=recv_sems.at[h],
                device_id=(right,),
                device_id_type=pl.DeviceIdType.MESH,
            )
            rdma.start()
            rdma.wait()

        for b in range(B):
            ctx_heads = []
            for hh in range(HQ):
                qh = qb[b * S_LOC:(b + 1) * S_LOC, hh * DH:(hh + 1) * DH]
                score_chunks = []
                for s in range(N_DEV):
                    kc = kv_buf[s, 0, b * S_LOC:(b + 1) * S_LOC,
                                hh * DH:(hh + 1) * DH]
                    score_chunks.append(
                        lax.dot_general(
                            qh, kc, (((1,), (1,)), ((), ())),
                            preferred_element_type=jnp.float32,
                        )
                    )
                scores = jnp.concatenate(score_chunks, axis=1) * 0.125
                m = jnp.max(scores, axis=1, keepdims=True)
                w = jnp.exp(scores - m)
                w = w / jnp.sum(w, axis=1, keepdims=True)
                wb = w.astype(jnp.bfloat16)
                acc = jnp.zeros((S_LOC, DH), jnp.float32)
                for s in range(N_DEV):
                    vc = kv_buf[s, 1, b * S_LOC:(b + 1) * S_LOC,
                                hh * DH:(hh + 1) * DH]
                    acc = acc + jnp.dot(
                        wb[:, s * S_LOC:(s + 1) * S_LOC], vc,
                        preferred_element_type=jnp.float32,
                    )
                ctx_heads.append(acc)
            ctx_b = jnp.concatenate(ctx_heads, axis=1).astype(jnp.bfloat16)
            out_ref[b] = jnp.dot(
                ctx_b, wo_ref[...].astype(jnp.bfloat16),
                preferred_element_type=jnp.float32,
            )

    return pl.pallas_call(
        body,
        out_shape=jax.ShapeDtypeStruct((B, S_LOC, D), jnp.float32),
        in_specs=[pl.BlockSpec(memory_space=pltpu.VMEM)] * 7,
        out_specs=pl.BlockSpec(memory_space=pltpu.VMEM),
        scratch_shapes=[
            pltpu.VMEM((N_DEV, 2, R, HD), jnp.bfloat16),
            pltpu.SemaphoreType.DMA((N_DEV - 1,)),
            pltpu.SemaphoreType.DMA((N_DEV - 1,)),
        ],
        compiler_params=pltpu.CompilerParams(collective_id=0),
    )(x, Wq, Wk, Wv, Wo, cos_full, sin_full)


# device time: 36959 ns/iter; 1.2069x vs baseline; 1.2069x over previous
import functools

import jax
import jax.numpy as jnp
from jax import lax
from jax.experimental import pallas as pl
from jax.experimental.pallas import tpu as pltpu

N_DEV = 8
B = 2
S_LOC = 128
D = 512
HQ = 4
DH = 64
HD = HQ * DH
R = B * S_LOC


def kernel(x, Wq, Wk, Wv, Wo):
    my = lax.axis_index("i")

    pos = (my * S_LOC).astype(jnp.float32) + jnp.arange(S_LOC, dtype=jnp.float32)
    inv = 1.0 / (10000.0 ** (jnp.arange(0, DH, 2, dtype=jnp.float32) / DH))
    ang = pos[:, None] * inv[None, :]
    cos = jnp.repeat(jnp.cos(ang), 2, axis=1)
    sin = jnp.repeat(jnp.sin(ang), 2, axis=1)
    cos_full = jnp.tile(cos, (B, HQ))
    sin_full = jnp.tile(sin, (B, HQ))

    def body(x_ref, wq_ref, wk_ref, wv_ref, wo_ref, cos_ref, sin_ref,
             out_ref, kv_buf, send_sems, recv_sems):
        me = lax.axis_index("i")
        partners = [me ^ 1, me ^ 3, me ^ 4]

        barrier_sem = pltpu.get_barrier_semaphore()
        for nbr in partners:
            pl.semaphore_signal(
                barrier_sem, inc=1,
                device_id=(nbr,), device_id_type=pl.DeviceIdType.MESH,
            )
        pl.semaphore_wait(barrier_sem, len(partners))

        xf = jnp.concatenate([x_ref[0], x_ref[1]], axis=0).astype(jnp.bfloat16)
        wk = wk_ref[...].astype(jnp.bfloat16)
        wv = wv_ref[...].astype(jnp.bfloat16)
        k = jnp.dot(xf, wk, preferred_element_type=jnp.float32)
        v = jnp.dot(xf, wv, preferred_element_type=jnp.float32)

        row = lax.broadcasted_iota(jnp.int32, (HD, HD), 0)
        col = lax.broadcasted_iota(jnp.int32, (HD, HD), 1)
        even = (col % 2) == 0
        r_mat = jnp.where(even & (row == col + 1), -1.0, 0.0) + jnp.where(
            (~even) & (row == col - 1), 1.0, 0.0
        )
        cosf = cos_ref[...]
        sinf = sin_ref[...]
        k_rot = k * cosf + jnp.dot(k, r_mat, preferred_element_type=jnp.float32) * sinf

        kv_buf[me, 0] = k_rot.astype(jnp.bfloat16)
        kv_buf[me, 1] = v.astype(jnp.bfloat16)

        def start_stage(st):
            size = 1 << st
            blk = me & ~jnp.int32(size - 1)
            rdma = pltpu.make_async_remote_copy(
                src_ref=kv_buf.at[pl.ds(blk, size)],
                dst_ref=kv_buf.at[pl.ds(blk, size)],
                send_sem=send_sems.at[st],
                recv_sem=recv_sems.at[st],
                device_id=(partners[st],),
                device_id_type=pl.DeviceIdType.MESH,
            )
            rdma.start()
            return rdma

        st0 = start_stage(0)
        wq = wq_ref[...].astype(jnp.bfloat16)
        q = jnp.dot(xf, wq, preferred_element_type=jnp.float32)
        q_rot = q * cosf + jnp.dot(q, r_mat, preferred_element_type=jnp.float32) * sinf
        qb = q_rot.astype(jnp.bfloat16)
        st0.wait()
        st1 = start_stage(1)
        st1.wait()
        st2 = start_stage(2)
        st2.wait()

        for b in range(B):
            ctx_heads = []
            for hh in range(HQ):
                qh = qb[b * S_LOC:(b + 1) * S_LOC, hh * DH:(hh + 1) * DH]
                score_chunks = []
                for s in range(N_DEV):
                    kc = kv_buf[s, 0, b * S_LOC:(b + 1) * S_LOC,
                                hh * DH:(hh + 1) * DH]
                    score_chunks.append(
                        lax.dot_general(
                            qh, kc, (((1,), (1,)), ((), ())),
                            preferred_element_type=jnp.float32,
                        )
                    )
                scores = jnp.concatenate(score_chunks, axis=1) * 0.125
                m = jnp.max(scores, axis=1, keepdims=True)
                w = jnp.exp(scores - m)
                w = w / jnp.sum(w, axis=1, keepdims=True)
                wb = w.astype(jnp.bfloat16)
                acc = jnp.zeros((S_LOC, DH), jnp.float32)
                for s in range(N_DEV):
                    vc = kv_buf[s, 1, b * S_LOC:(b + 1) * S_LOC,
                                hh * DH:(hh + 1) * DH]
                    acc = acc + jnp.dot(
                        wb[:, s * S_LOC:(s + 1) * S_LOC], vc,
                        preferred_element_type=jnp.float32,
                    )
                ctx_heads.append(acc)
            ctx_b = jnp.concatenate(ctx_heads, axis=1).astype(jnp.bfloat16)
            out_ref[b] = jnp.dot(
                ctx_b, wo_ref[...].astype(jnp.bfloat16),
                preferred_element_type=jnp.float32,
            )

    return pl.pallas_call(
        body,
        out_shape=jax.ShapeDtypeStruct((B, S_LOC, D), jnp.float32),
        in_specs=[pl.BlockSpec(memory_space=pltpu.VMEM)] * 7,
        out_specs=pl.BlockSpec(memory_space=pltpu.VMEM),
        scratch_shapes=[
            pltpu.VMEM((N_DEV, 2, R, HD), jnp.bfloat16),
            pltpu.SemaphoreType.DMA((3,)),
            pltpu.SemaphoreType.DMA((3,)),
        ],
        compiler_params=pltpu.CompilerParams(collective_id=0),
    )(x, Wq, Wk, Wv, Wo, cos_full, sin_full)


# device time: 28589 ns/iter; 1.5602x vs baseline; 1.2928x over previous
import functools

import jax
import jax.numpy as jnp
from jax import lax
from jax.experimental import pallas as pl
from jax.experimental.pallas import tpu as pltpu

N_DEV = 8
B = 2
S_LOC = 128
D = 512
HQ = 4
DH = 64
HD = HQ * DH
R = B * S_LOC


def kernel(x, Wq, Wk, Wv, Wo):
    my = lax.axis_index("i")

    pos = (my * S_LOC).astype(jnp.float32) + jnp.arange(S_LOC, dtype=jnp.float32)
    inv = 1.0 / (10000.0 ** (jnp.arange(0, DH, 2, dtype=jnp.float32) / DH))
    ang = pos[:, None] * inv[None, :]
    cos = jnp.repeat(jnp.cos(ang), 2, axis=1)
    sin = jnp.repeat(jnp.sin(ang), 2, axis=1)
    cos_full = jnp.tile(cos, (B, HQ))
    sin_full = jnp.tile(sin, (B, HQ))

    def body(x_ref, wq_ref, wk_ref, wv_ref, wo_ref, cos_ref, sin_ref,
             out_ref, kv_buf, send_sems, recv_sems):
        me = lax.axis_index("i")
        partners = [me ^ 1, me ^ 3, me ^ 4]

        barrier_sem = pltpu.get_barrier_semaphore()
        for nbr in partners:
            pl.semaphore_signal(
                barrier_sem, inc=1,
                device_id=(nbr,), device_id_type=pl.DeviceIdType.MESH,
            )
        pl.semaphore_wait(barrier_sem, len(partners))

        xf = jnp.concatenate([x_ref[0], x_ref[1]], axis=0).astype(jnp.bfloat16)
        wk = wk_ref[...].astype(jnp.bfloat16)
        wv = wv_ref[...].astype(jnp.bfloat16)
        k = jnp.dot(xf, wk, preferred_element_type=jnp.float32)
        v = jnp.dot(xf, wv, preferred_element_type=jnp.float32)

        row = lax.broadcasted_iota(jnp.int32, (HD, HD), 0)
        col = lax.broadcasted_iota(jnp.int32, (HD, HD), 1)
        even = (col % 2) == 0
        r_mat = jnp.where(even & (row == col + 1), -1.0, 0.0) + jnp.where(
            (~even) & (row == col - 1), 1.0, 0.0
        )
        cosf = cos_ref[...]
        sinf = sin_ref[...]
        k_rot = k * cosf + jnp.dot(k, r_mat, preferred_element_type=jnp.float32) * sinf

        kv_buf[me, 0] = k_rot.astype(jnp.bfloat16)
        kv_buf[me, 1] = v.astype(jnp.bfloat16)

        def start_flow(idx, slot, size, partner):
            rdma = pltpu.make_async_remote_copy(
                src_ref=kv_buf.at[pl.ds(slot, size)],
                dst_ref=kv_buf.at[pl.ds(slot, size)],
                send_sem=send_sems.at[idx],
                recv_sem=recv_sems.at[idx],
                device_id=(partner,),
                device_id_type=pl.DeviceIdType.MESH,
            )
            rdma.start()
            return rdma

        r_own_x = start_flow(0, me, 1, partners[0])
        r_own_y = start_flow(1, me, 1, partners[1])
        r_own_z = start_flow(2, me, 1, partners[2])

        wq = wq_ref[...].astype(jnp.bfloat16)
        q = jnp.dot(xf, wq, preferred_element_type=jnp.float32)
        q_rot = q * cosf + jnp.dot(q, r_mat, preferred_element_type=jnp.float32) * sinf
        qb = q_rot.astype(jnp.bfloat16)

        r_own_x.wait_recv()
        r_fwd_y = start_flow(3, me ^ 1, 1, partners[1])
        r_fwd_z = start_flow(4, me ^ 1, 1, partners[2])
        r_own_y.wait_recv()
        r_fwd_y.wait_recv()
        r_blk_z = start_flow(5, (me & ~jnp.int32(1)) ^ 2, 2, partners[2])
        r_own_z.wait_recv()
        r_fwd_z.wait_recv()
        r_blk_z.wait_recv()
        for r in (r_own_x, r_own_y, r_own_z, r_fwd_y, r_fwd_z, r_blk_z):
            r.wait_send()

        for b in range(B):
            ctx_heads = []
            for hh in range(HQ):
                qh = qb[b * S_LOC:(b + 1) * S_LOC, hh * DH:(hh + 1) * DH]
                score_chunks = []
                for s in range(N_DEV):
                    kc = kv_buf[s, 0, b * S_LOC:(b + 1) * S_LOC,
                                hh * DH:(hh + 1) * DH]
                    score_chunks.append(
                        lax.dot_general(
                            qh, kc, (((1,), (1,)), ((), ())),
                            preferred_element_type=jnp.float32,
                        )
                    )
                scores = jnp.concatenate(score_chunks, axis=1) * 0.125
                m = jnp.max(scores, axis=1, keepdims=True)
                w = jnp.exp(scores - m)
                w = w / jnp.sum(w, axis=1, keepdims=True)
                wb = w.astype(jnp.bfloat16)
                acc = jnp.zeros((S_LOC, DH), jnp.float32)
                for s in range(N_DEV):
                    vc = kv_buf[s, 1, b * S_LOC:(b + 1) * S_LOC,
                                hh * DH:(hh + 1) * DH]
                    acc = acc + jnp.dot(
                        wb[:, s * S_LOC:(s + 1) * S_LOC], vc,
                        preferred_element_type=jnp.float32,
                    )
                ctx_heads.append(acc)
            ctx_b = jnp.concatenate(ctx_heads, axis=1).astype(jnp.bfloat16)
            out_ref[b] = jnp.dot(
                ctx_b, wo_ref[...].astype(jnp.bfloat16),
                preferred_element_type=jnp.float32,
            )

    return pl.pallas_call(
        body,
        out_shape=jax.ShapeDtypeStruct((B, S_LOC, D), jnp.float32),
        in_specs=[pl.BlockSpec(memory_space=pltpu.VMEM)] * 7,
        out_specs=pl.BlockSpec(memory_space=pltpu.VMEM),
        scratch_shapes=[
            pltpu.VMEM((N_DEV, 2, R, HD), jnp.bfloat16),
            pltpu.SemaphoreType.DMA((6,)),
            pltpu.SemaphoreType.DMA((6,)),
        ],
        compiler_params=pltpu.CompilerParams(collective_id=0),
    )(x, Wq, Wk, Wv, Wo, cos_full, sin_full)


# device time: 26461 ns/iter; 1.6857x vs baseline; 1.0804x over previous
import functools

import jax
import jax.numpy as jnp
from jax import lax
from jax.experimental import pallas as pl
from jax.experimental.pallas import tpu as pltpu

N_DEV = 8
B = 2
S_LOC = 128
D = 512
HQ = 4
DH = 64
HD = HQ * DH
R = B * S_LOC


def kernel(x, Wq, Wk, Wv, Wo):
    my = lax.axis_index("i")

    pos = (my * S_LOC).astype(jnp.float32) + jnp.arange(S_LOC, dtype=jnp.float32)
    inv = 1.0 / (10000.0 ** (jnp.arange(0, DH, 2, dtype=jnp.float32) / DH))
    ang = pos[:, None] * inv[None, :]
    cos = jnp.repeat(jnp.cos(ang), 2, axis=1)
    sin = jnp.repeat(jnp.sin(ang), 2, axis=1)
    cos_full = jnp.tile(cos, (B, HQ))
    sin_full = jnp.tile(sin, (B, HQ))

    def body(x_ref, wq_ref, wk_ref, wv_ref, wo_ref, cos_ref, sin_ref,
             out_ref, kv_buf, scores_buf, send_sems, recv_sems):
        me = lax.axis_index("i")
        partners = [me ^ 1, me ^ 3, me ^ 4]

        barrier_sem = pltpu.get_barrier_semaphore()
        for nbr in partners:
            pl.semaphore_signal(
                barrier_sem, inc=1,
                device_id=(nbr,), device_id_type=pl.DeviceIdType.MESH,
            )
        pl.semaphore_wait(barrier_sem, len(partners))

        xf = jnp.concatenate([x_ref[0], x_ref[1]], axis=0).astype(jnp.bfloat16)
        wk = wk_ref[...].astype(jnp.bfloat16)
        wv = wv_ref[...].astype(jnp.bfloat16)
        k = jnp.dot(xf, wk, preferred_element_type=jnp.float32)
        v = jnp.dot(xf, wv, preferred_element_type=jnp.float32)

        row = lax.broadcasted_iota(jnp.int32, (HD, HD), 0)
        col = lax.broadcasted_iota(jnp.int32, (HD, HD), 1)
        even = (col % 2) == 0
        r_mat = jnp.where(even & (row == col + 1), -1.0, 0.0) + jnp.where(
            (~even) & (row == col - 1), 1.0, 0.0
        )
        cosf = cos_ref[...]
        sinf = sin_ref[...]
        k_rot = k * cosf + jnp.dot(k, r_mat, preferred_element_type=jnp.float32) * sinf

        kv_buf[me, 0] = k_rot.astype(jnp.bfloat16)
        kv_buf[me, 1] = v.astype(jnp.bfloat16)

        def start_flow(idx, slot, size, partner):
            rdma = pltpu.make_async_remote_copy(
                src_ref=kv_buf.at[pl.ds(slot, size)],
                dst_ref=kv_buf.at[pl.ds(slot, size)],
                send_sem=send_sems.at[idx],
                recv_sem=recv_sems.at[idx],
                device_id=(partner,),
                device_id_type=pl.DeviceIdType.MESH,
            )
            rdma.start()
            return rdma

        def scores_for(slot):
            for b in range(B):
                for hh in range(HQ):
                    qh = qb[b * S_LOC:(b + 1) * S_LOC, hh * DH:(hh + 1) * DH]
                    kc = kv_buf[slot, 0, b * S_LOC:(b + 1) * S_LOC,
                                hh * DH:(hh + 1) * DH]
                    sc = lax.dot_general(
                        qh, kc, (((1,), (1,)), ((), ())),
                        preferred_element_type=jnp.float32,
                    )
                    scores_buf[b, hh, :, pl.ds(slot * S_LOC, S_LOC)] = sc * 0.125

        r_own_x = start_flow(0, me, 1, partners[0])
        r_own_y = start_flow(1, me, 1, partners[1])
        r_own_z = start_flow(2, me, 1, partners[2])

        wq = wq_ref[...].astype(jnp.bfloat16)
        q = jnp.dot(xf, wq, preferred_element_type=jnp.float32)
        q_rot = q * cosf + jnp.dot(q, r_mat, preferred_element_type=jnp.float32) * sinf
        qb = q_rot.astype(jnp.bfloat16)
        scores_for(me)

        r_own_x.wait_recv()
        r_fwd_y = start_flow(3, me ^ 1, 1, partners[1])
        r_fwd_z = start_flow(4, me ^ 1, 1, partners[2])
        scores_for(me ^ 1)
        r_own_y.wait_recv()
        r_fwd_y.wait_recv()
        r_blk_z = start_flow(5, (me & ~jnp.int32(1)) ^ 2, 2, partners[2])
        scores_for(me ^ 3)
        scores_for(me ^ 2)
        r_own_z.wait_recv()
        scores_for(me ^ 4)
        r_fwd_z.wait_recv()
        scores_for(me ^ 5)
        r_blk_z.wait_recv()
        scores_for(me ^ 6)
        scores_for(me ^ 7)
        for r in (r_own_x, r_own_y, r_own_z, r_fwd_y, r_fwd_z, r_blk_z):
            r.wait_send()

        for b in range(B):
            ctx_heads = []
            for hh in range(HQ):
                scores = scores_buf[b, hh]
                m = jnp.max(scores, axis=1, keepdims=True)
                w = jnp.exp(scores - m)
                w = w / jnp.sum(w, axis=1, keepdims=True)
                wb = w.astype(jnp.bfloat16)
                acc = jnp.zeros((S_LOC, DH), jnp.float32)
                for s in range(N_DEV):
                    vc = kv_buf[s, 1, b * S_LOC:(b + 1) * S_LOC,
                                hh * DH:(hh + 1) * DH]
                    acc = acc + jnp.dot(
                        wb[:, s * S_LOC:(s + 1) * S_LOC], vc,
                        preferred_element_type=jnp.float32,
                    )
                ctx_heads.append(acc)
            ctx_b = jnp.concatenate(ctx_heads, axis=1).astype(jnp.bfloat16)
            out_ref[b] = jnp.dot(
                ctx_b, wo_ref[...].astype(jnp.bfloat16),
                preferred_element_type=jnp.float32,
            )

    return pl.pallas_call(
        body,
        out_shape=jax.ShapeDtypeStruct((B, S_LOC, D), jnp.float32),
        in_specs=[pl.BlockSpec(memory_space=pltpu.VMEM)] * 7,
        out_specs=pl.BlockSpec(memory_space=pltpu.VMEM),
        scratch_shapes=[
            pltpu.VMEM((N_DEV, 2, R, HD), jnp.bfloat16),
            pltpu.VMEM((B, HQ, S_LOC, N_DEV * S_LOC), jnp.float32),
            pltpu.SemaphoreType.DMA((6,)),
            pltpu.SemaphoreType.DMA((6,)),
        ],
        compiler_params=pltpu.CompilerParams(collective_id=0),
    )(x, Wq, Wk, Wv, Wo, cos_full, sin_full)
